# Initial kernel scaffold; baseline (speedup 1.0000x reference)
import functools

import jax
import jax.numpy as jnp
from jax import lax
from jax.experimental import pallas as pl
from jax.experimental.pallas import tpu as pltpu
from jax.experimental.pallas import tpu_sc as plsc

B = 16384
L = 20
VOCAB = 30522
WORD_DIM = 300
HIDDEN = 128

PROJ_BLK = 512
VPAD = 30528
DUMMY = VOCAB
PROJ_GRID = (VPAD + PROJ_BLK - 1) // PROJ_BLK

PREP_BLK = 2048

_info = plsc.get_sparse_core_info()
NC = _info.num_cores
NS = _info.num_subcores
LANES = _info.num_lanes
NW = NC * NS
RPW = B // NW
CH = 4
GRP = 64
CHG = GRP // CH
NGRP = RPW // GRP


def _proj_body(emb_ref, w_ref, b_ref, out_ref):
    i = pl.program_id(0)
    rows = i * PROJ_BLK + lax.broadcasted_iota(jnp.int32, (PROJ_BLK, 1), 0)
    valid = rows < VOCAB
    x = jnp.where(valid, emb_ref[...], 0.0)
    y = jnp.dot(x, w_ref[...], preferred_element_type=jnp.float32)
    y = jnp.maximum(y + b_ref[...], 0.0)
    out_ref[...] = jnp.where(valid, y, 0.0)


_project = pl.pallas_call(
    _proj_body,
    grid=(PROJ_GRID,),
    in_specs=[
        pl.BlockSpec((PROJ_BLK, WORD_DIM), lambda i: (i, 0)),
        pl.BlockSpec((WORD_DIM, HIDDEN), lambda i: (0, 0)),
        pl.BlockSpec((1, HIDDEN), lambda i: (0, 0)),
    ],
    out_specs=pl.BlockSpec((PROJ_BLK, HIDDEN), lambda i: (i, 0)),
    out_shape=jax.ShapeDtypeStruct((VPAD, HIDDEN), jnp.float32),
)


def _prep_body(tok_ref, remap_ref, inv_ref):
    t = tok_ref[...]
    m = t > 2
    remap_ref[...] = jnp.where(m, t, DUMMY)
    cnt = jnp.sum(m.astype(jnp.float32), axis=1, keepdims=True)
    inv_ref[...] = 1.0 / (cnt + 1e-10)


_prep = pl.pallas_call(
    _prep_body,
    grid=(B // PREP_BLK,),
    in_specs=[pl.BlockSpec((PREP_BLK, L), lambda i: (i, 0))],
    out_specs=[
        pl.BlockSpec((PREP_BLK, L), lambda i: (i, 0)),
        pl.BlockSpec((PREP_BLK, 1), lambda i: (i, 0)),
    ],
    out_shape=[
        jax.ShapeDtypeStruct((B, L), jnp.int32),
        jax.ShapeDtypeStruct((B, 1), jnp.float32),
    ],
)


@functools.partial(
    pl.kernel,
    out_type=jax.ShapeDtypeStruct((B, HIDDEN), jnp.float32),
    mesh=plsc.VectorSubcoreMesh(core_axis_name="c", subcore_axis_name="s"),
    scratch_types=[
        pltpu.VMEM((RPW * L,), jnp.int32),
        pltpu.VMEM((RPW,), jnp.float32),
        pltpu.VMEM((CH * L, HIDDEN), jnp.float32),
        pltpu.VMEM((GRP, HIDDEN), jnp.float32),
        pltpu.SemaphoreType.DMA,
    ],
)
def _sc_pool(table_hbm, tok_hbm, inv_hbm, out_hbm, tok_v, inv_v, rows_v,
             out_v, sem):
    wid = lax.axis_index("s") * NC + lax.axis_index("c")
    base = wid * RPW
    pltpu.sync_copy(tok_hbm.at[pl.ds(base * L, RPW * L)], tok_v)
    pltpu.sync_copy(inv_hbm.at[pl.ds(base, RPW)], inv_v)

    def group(g, carry):
        def chunk(c, carry):
            row0 = g * GRP + c * CH
            off = pl.multiple_of(row0 * L, 8)
            pltpu.async_copy(
                table_hbm.at[tok_v.at[pl.ds(off, CH * L)]], rows_v, sem
            ).wait()
            for r in range(CH):
                inv_vec = plsc.load_gather(
                    inv_v, [jnp.full((LANES,), row0 + r, jnp.int32)]
                )
                for h in range(HIDDEN // LANES):
                    sl = pl.ds(h * LANES, LANES)
                    acc = rows_v[r * L, sl]
                    for j in range(1, L):
                        acc = acc + rows_v[r * L + j, sl]
                    out_v[c * CH + r, sl] = acc * inv_vec
            return carry

        lax.fori_loop(0, CHG, chunk, 0)
        pltpu.sync_copy(out_v, out_hbm.at[pl.ds(base + g * GRP, GRP)])
        return carry

    lax.fori_loop(0, NGRP, group, 0)


def kernel(tokens, word_embeddings, W_fc, b_fc):
    table = _project(word_embeddings, W_fc, b_fc.reshape(1, HIDDEN))
    remap, inv = _prep(tokens)
    return _sc_pool(table, remap.reshape(B * L), inv.reshape(B))


# speedup vs baseline: 5.0630x; 5.0630x over previous
import functools

import jax
import jax.numpy as jnp
from jax import lax
from jax.experimental import pallas as pl
from jax.experimental.pallas import tpu as pltpu
from jax.experimental.pallas import tpu_sc as plsc

B = 16384
L = 20
VOCAB = 30522
WORD_DIM = 300
HIDDEN = 128

PROJ_BLK = 512
VPAD = 30528
DUMMY = VOCAB
PROJ_GRID = (VPAD + PROJ_BLK - 1) // PROJ_BLK

PREP_BLK = 2048

NC = 2
NS = 16
LANES = 16
NW = NC * NS
RPW = B // NW
CH = 4
GRP = 64
CHG = GRP // CH
NGRP = RPW // GRP


def _proj_body(emb_ref, w_ref, b_ref, out_ref):
    i = pl.program_id(0)
    rows = i * PROJ_BLK + lax.broadcasted_iota(jnp.int32, (PROJ_BLK, 1), 0)
    valid = rows < VOCAB
    x = jnp.where(valid, emb_ref[...], 0.0)
    y = jnp.dot(x, w_ref[...], preferred_element_type=jnp.float32)
    y = jnp.maximum(y + b_ref[...], 0.0)
    out_ref[...] = jnp.where(valid, y, 0.0)


_project = pl.pallas_call(
    _proj_body,
    grid=(PROJ_GRID,),
    in_specs=[
        pl.BlockSpec((PROJ_BLK, WORD_DIM), lambda i: (i, 0)),
        pl.BlockSpec((WORD_DIM, HIDDEN), lambda i: (0, 0)),
        pl.BlockSpec((1, HIDDEN), lambda i: (0, 0)),
    ],
    out_specs=pl.BlockSpec((PROJ_BLK, HIDDEN), lambda i: (i, 0)),
    out_shape=jax.ShapeDtypeStruct((VPAD, HIDDEN), jnp.float32),
)


def _prep_body(tok_ref, remap_ref, inv_ref):
    t = tok_ref[...]
    m = t > 2
    remap_ref[...] = jnp.where(m, t, DUMMY)
    cnt = jnp.sum(m.astype(jnp.float32), axis=1, keepdims=True)
    inv_ref[...] = jnp.broadcast_to(1.0 / (cnt + 1e-10), (PREP_BLK, LANES))


_prep = pl.pallas_call(
    _prep_body,
    grid=(B // PREP_BLK,),
    in_specs=[pl.BlockSpec((PREP_BLK, L), lambda i: (i, 0))],
    out_specs=[
        pl.BlockSpec((PREP_BLK, L), lambda i: (i, 0)),
        pl.BlockSpec((PREP_BLK, LANES), lambda i: (i, 0)),
    ],
    out_shape=[
        jax.ShapeDtypeStruct((B, L), jnp.int32),
        jax.ShapeDtypeStruct((B, LANES), jnp.float32),
    ],
)


@functools.partial(
    pl.kernel,
    out_type=jax.ShapeDtypeStruct((B, HIDDEN), jnp.float32),
    mesh=plsc.VectorSubcoreMesh(core_axis_name="c", subcore_axis_name="s"),
    scratch_types=[
        pltpu.VMEM((RPW * L,), jnp.int32),
        pltpu.VMEM((RPW, LANES), jnp.float32),
        pltpu.VMEM((CH * L, HIDDEN), jnp.float32),
        pltpu.VMEM((GRP, HIDDEN), jnp.float32),
        pltpu.SemaphoreType.DMA,
    ],
)
def _sc_pool(table_hbm, tok_hbm, inv_hbm, out_hbm, tok_v, inv_v, rows_v,
             out_v, sem):
    wid = lax.axis_index("s") * NC + lax.axis_index("c")
    base = wid * RPW
    pltpu.sync_copy(tok_hbm.at[pl.ds(base * L, RPW * L)], tok_v)
    pltpu.sync_copy(inv_hbm.at[pl.ds(base, RPW)], inv_v)

    def group(g, carry):
        def chunk(c, carry):
            row0 = g * GRP + c * CH
            off = pl.multiple_of(row0 * L, 8)
            pltpu.async_copy(
                table_hbm.at[tok_v.at[pl.ds(off, CH * L)]], rows_v, sem
            ).wait()
            for r in range(CH):
                inv_vec = inv_v[row0 + r, :]
                for h in range(HIDDEN // LANES):
                    sl = pl.ds(h * LANES, LANES)
                    acc = rows_v[r * L, sl]
                    for j in range(1, L):
                        acc = acc + rows_v[r * L + j, sl]
                    out_v[c * CH + r, sl] = acc * inv_vec
            return carry

        lax.fori_loop(0, CHG, chunk, 0)
        pltpu.sync_copy(out_v, out_hbm.at[pl.ds(base + g * GRP, GRP)])
        return carry

    lax.fori_loop(0, NGRP, group, 0)


def kernel(tokens, word_embeddings, W_fc, b_fc):
    table = _project(word_embeddings, W_fc, b_fc.reshape(1, HIDDEN))
    remap, inv = _prep(tokens)
    return _sc_pool(table, remap.reshape(B * L), inv)
